# Initial kernel scaffold; baseline (speedup 1.0000x reference)
import functools

import jax
import jax.numpy as jnp
from jax import lax
from jax.experimental import pallas as pl
from jax.experimental.pallas import tpu as pltpu
from jax.experimental.pallas import tpu_sc as plsc

_NW = 32
_LANES = 16


def _sc_max_gather(Np, per_w, CH, K, C):
    n_chunks = per_w // CH
    G = CH * K
    slices = []
    off = 0
    while off < G:
        sz = min(128, G - off)
        slices.append((off, sz))
        off += sz
    mesh = plsc.VectorSubcoreMesh(core_axis_name="c", subcore_axis_name="s")

    @functools.partial(
        pl.kernel,
        mesh=mesh,
        out_type=jax.ShapeDtypeStruct((Np, C), jnp.float32),
        scratch_types=[
            pltpu.VMEM((G,), jnp.int32),
            pltpu.VMEM((G, C), jnp.float32),
            pltpu.VMEM((CH, C), jnp.float32),
            pltpu.SemaphoreType.DMA,
        ],
    )
    def sc_fn(x_hbm, idx_hbm, out_hbm, idx_v, rows_v, out_v, sem):
        wid = lax.axis_index("s") * 2 + lax.axis_index("c")
        base = wid * per_w

        def chunk(it, carry):
            nb = base + it * CH
            pltpu.sync_copy(idx_hbm.at[pl.ds(nb * K, G)], idx_v)
            cps = [
                pltpu.async_copy(
                    x_hbm.at[idx_v.at[pl.ds(o, s)]],
                    rows_v.at[pl.ds(o, s)],
                    sem,
                )
                for (o, s) in slices
            ]
            for cp in cps:
                cp.wait()

            def node(ni, c2):
                r0 = ni * K
                for cg in range(C // _LANES):
                    sl = pl.ds(cg * _LANES, _LANES)
                    m = rows_v[r0, sl]
                    for j in range(1, K):
                        m = jnp.maximum(m, rows_v[r0 + j, sl])
                    out_v[ni, sl] = m
                return c2

            lax.fori_loop(0, CH, node, 0)
            pltpu.sync_copy(out_v, out_hbm.at[pl.ds(nb, CH)])
            return carry

        lax.fori_loop(0, n_chunks, chunk, 0)

    return sc_fn


def _tc_mlp(x_cn, aggr, Wd, W2, b2, NBLK):
    C, N = x_cn.shape
    C_OUT = Wd.shape[0]
    grid = N // NBLK

    def tc_fn(x_ref, a_ref, wd_ref, w2_ref, b_ref, o_ref):
        mm1 = lax.dot_general(
            wd_ref[...], x_ref[...], (((1,), (0,)), ((), ())),
            preferred_element_type=jnp.float32)
        mm2 = lax.dot_general(
            w2_ref[...], a_ref[...], (((1,), (1,)), ((), ())),
            preferred_element_type=jnp.float32)
        o_ref[...] = jnp.maximum(mm1 + mm2 + b_ref[...], 0.0)

    return pl.pallas_call(
        tc_fn,
        grid=(grid,),
        in_specs=[
            pl.BlockSpec((C, NBLK), lambda i: (0, i)),
            pl.BlockSpec((NBLK, C), lambda i: (i, 0)),
            pl.BlockSpec((C_OUT, C), lambda i: (0, 0)),
            pl.BlockSpec((C_OUT, C), lambda i: (0, 0)),
            pl.BlockSpec((C_OUT, 1), lambda i: (0, 0)),
        ],
        out_specs=pl.BlockSpec((C_OUT, NBLK), lambda i: (0, i)),
        out_shape=jax.ShapeDtypeStruct((C_OUT, N), jnp.float32),
    )(x_cn, aggr, Wd, W2, b2)


def kernel(x, edge_index, W, b):
    B, C, N, _ = x.shape
    K = edge_index.shape[-1]
    C_OUT = W.shape[0]

    x_cn = x[0, :, :, 0]
    x_flat = jnp.transpose(x_cn)
    idx = edge_index[0, 0]

    per_w = (-(-N // _NW) + 7) // 8 * 8
    Np = per_w * _NW
    idx_flat = jnp.pad(jnp.reshape(idx, (-1,)), (0, Np * K - N * K))

    CH = 56
    while per_w % CH:
        CH -= 8
    aggr = _sc_max_gather(Np, per_w, CH, K, C)(x_flat, idx_flat)

    W1, W2 = W[:, :C], W[:, C:]
    Wd = W1 - W2
    out = _tc_mlp(x_cn, aggr, Wd, W2, jnp.reshape(b, (C_OUT, 1)), 2000)
    return out[None, :, :, None]


# speedup vs baseline: 19.4998x; 19.4998x over previous
import functools

import jax
import jax.numpy as jnp
from jax import lax
from jax.experimental import pallas as pl
from jax.experimental.pallas import tpu as pltpu
from jax.experimental.pallas import tpu_sc as plsc

_NW = 32
_LANES = 16


def _sc_max_gather(Np, per_w, CH, K, C):
    n_chunks = per_w // CH
    G = CH * K
    slices = []
    off = 0
    while off < G:
        sz = min(128, G - off)
        slices.append((off, sz))
        off += sz
    mesh = plsc.VectorSubcoreMesh(core_axis_name="c", subcore_axis_name="s")

    @functools.partial(
        pl.kernel,
        mesh=mesh,
        out_type=jax.ShapeDtypeStruct((Np, C), jnp.float32),
        scratch_types=[
            pltpu.VMEM((G,), jnp.int32),
            pltpu.VMEM((G, C), jnp.float32),
            pltpu.VMEM((CH, C), jnp.float32),
            pltpu.SemaphoreType.DMA,
        ],
    )
    def sc_fn(x_hbm, idx_hbm, out_hbm, idx_v, rows_v, out_v, sem):
        wid = lax.axis_index("s") * 2 + lax.axis_index("c")
        base = wid * per_w

        def chunk(it, carry):
            nb = base + it * CH
            pltpu.sync_copy(idx_hbm.at[pl.ds(nb * K, G)], idx_v)
            cps = [
                pltpu.async_copy(
                    x_hbm.at[idx_v.at[pl.ds(o, s)]],
                    rows_v.at[pl.ds(o, s)],
                    sem,
                )
                for (o, s) in slices
            ]
            for cp in cps:
                cp.wait()

            def node(ni, c2):
                r0 = ni * K
                for cg in range(C // _LANES):
                    sl = pl.ds(cg * _LANES, _LANES)
                    m = rows_v[r0, sl]
                    for j in range(1, K):
                        m = jnp.maximum(m, rows_v[r0 + j, sl])
                    out_v[ni, sl] = m
                return c2

            lax.fori_loop(0, CH, node, 0)
            pltpu.sync_copy(out_v, out_hbm.at[pl.ds(nb, CH)])
            return carry

        lax.fori_loop(0, n_chunks, chunk, 0)

    return sc_fn


def _tc_mlp(x_cn, aggr, Wd, W2, b2, NBLK):
    C, N = x_cn.shape
    C_OUT = Wd.shape[0]
    grid = pl.cdiv(N, NBLK)

    def tc_fn(x_ref, a_ref, wd_ref, w2_ref, b_ref, o_ref):
        mm1 = lax.dot_general(
            wd_ref[...], x_ref[...], (((1,), (0,)), ((), ())),
            preferred_element_type=jnp.float32)
        mm2 = lax.dot_general(
            w2_ref[...], a_ref[...], (((1,), (1,)), ((), ())),
            preferred_element_type=jnp.float32)
        o_ref[...] = jnp.maximum(mm1 + mm2 + b_ref[...], 0.0)

    return pl.pallas_call(
        tc_fn,
        grid=(grid,),
        in_specs=[
            pl.BlockSpec((C, NBLK), lambda i: (0, i)),
            pl.BlockSpec((NBLK, C), lambda i: (i, 0)),
            pl.BlockSpec((C_OUT, C), lambda i: (0, 0)),
            pl.BlockSpec((C_OUT, C), lambda i: (0, 0)),
            pl.BlockSpec((C_OUT, 1), lambda i: (0, 0)),
        ],
        out_specs=pl.BlockSpec((C_OUT, NBLK), lambda i: (0, i)),
        out_shape=jax.ShapeDtypeStruct((C_OUT, N), jnp.float32),
    )(x_cn, aggr, Wd, W2, b2)


def kernel(x, edge_index, W, b):
    B, C, N, _ = x.shape
    K = edge_index.shape[-1]
    C_OUT = W.shape[0]

    x_cn = x[0, :, :, 0]
    x_flat = jnp.transpose(x_cn)
    idx = edge_index[0, 0]

    per_w = (-(-N // _NW) + 7) // 8 * 8
    Np = per_w * _NW
    idx_flat = jnp.pad(jnp.reshape(idx, (-1,)), (0, Np * K - N * K))

    CH = 56
    while per_w % CH:
        CH -= 8
    aggr = _sc_max_gather(Np, per_w, CH, K, C)(x_flat, idx_flat)

    W1, W2 = W[:, :C], W[:, C:]
    Wd = W1 - W2
    out = _tc_mlp(x_cn, aggr, Wd, W2, jnp.reshape(b, (C_OUT, 1)), 2048)
    return out[None, :, :, None]
